# Initial kernel scaffold; baseline (speedup 1.0000x reference)
import jax
import jax.numpy as jnp
from jax.experimental import pallas as pl


def kernel(hidden_states, Wq, Wk, Wv, Wo, Wq_idx, Wk_idx):
    raise NotImplementedError("write your pallas kernel here")


# speedup vs baseline: 48.2780x; 48.2780x over previous
import functools
import math

import jax
import jax.numpy as jnp
from jax.experimental import pallas as pl
from jax.experimental.pallas import tpu as pltpu

D_MODEL_ = 1024
N_HEADS_ = 16
N_SEL_ = 64
IDX_DIM_ = 32
D_HEAD_ = D_MODEL_ // N_HEADS_

_INT_MIN = -2147483648
_INT_MAX = 2147483647


def _sortable_key(x):
    bits = jax.lax.bitcast_convert_type(x, jnp.int32)
    return jnp.where(bits < 0, bits ^ jnp.int32(0x7FFFFFFF), bits)


def _floor_avg(a, b):
    return (a & b) + ((a ^ b) >> 1)


def _attn_body(x_ref, wq_ref, wk_ref, wv_ref, wo_ref, wqi_ref, wki_ref,
               out_ref, u_scr):
    h = pl.program_id(0)
    f32 = jnp.float32

    x = x_ref[...]
    q = jnp.dot(x, wq_ref[0], preferred_element_type=f32)
    k = jnp.dot(x, wk_ref[0], preferred_element_type=f32)
    v = jnp.dot(x, wv_ref[0], preferred_element_type=f32)

    qi = jnp.dot(q, wqi_ref[...], preferred_element_type=f32)
    ki = jnp.dot(k, wki_ref[...], preferred_element_type=f32)
    si = jnp.dot(qi, ki.T, preferred_element_type=f32)
    si = si * f32(1.0 / math.sqrt(IDX_DIM_))

    u_scr[...] = _sortable_key(si)

    t = si.shape[0]
    lo0 = jnp.full((t, 1), _INT_MIN, dtype=jnp.int32)
    hi0 = jnp.full((t, 1), _INT_MAX, dtype=jnp.int32)

    def bisect(_, carry):
        lo, hi = carry
        mid = _floor_avg(lo, hi)
        cnt = jnp.sum((u_scr[...] >= mid).astype(jnp.int32), axis=1,
                      keepdims=True)
        ge = cnt >= N_SEL_
        return jnp.where(ge, mid, lo), jnp.where(ge, hi, mid)

    tau, _ = jax.lax.fori_loop(0, 32, bisect, (lo0, hi0))
    mask = u_scr[...] >= tau

    s = jnp.dot(q, k.T, preferred_element_type=f32) * f32(1.0 / math.sqrt(D_HEAD_))
    s = jnp.where(mask, s, f32(-jnp.inf))
    m = jnp.max(s, axis=1, keepdims=True)
    p = jnp.exp(s - m)
    p = p / jnp.sum(p, axis=1, keepdims=True)

    o = jnp.dot(p, v, preferred_element_type=f32)
    contrib = jnp.dot(o, wo_ref[0], preferred_element_type=f32)

    @pl.when(h == 0)
    def _():
        out_ref[...] = contrib

    @pl.when(h != 0)
    def _():
        out_ref[...] += contrib


@jax.jit
def kernel(hidden_states, Wq, Wk, Wv, Wo, Wq_idx, Wk_idx):
    b, t, d = hidden_states.shape
    x = hidden_states.reshape(t, d)
    dh = D_HEAD_

    wq_h = Wq.reshape(d, N_HEADS_, dh).transpose(1, 0, 2)
    wk_h = Wk.reshape(d, N_HEADS_, dh).transpose(1, 0, 2)
    wv_h = Wv.reshape(d, N_HEADS_, dh).transpose(1, 0, 2)
    wo_h = Wo.reshape(N_HEADS_, dh, d)

    out = pl.pallas_call(
        _attn_body,
        grid=(N_HEADS_,),
        in_specs=[
            pl.BlockSpec((t, d), lambda h: (0, 0)),
            pl.BlockSpec((1, d, dh), lambda h: (h, 0, 0)),
            pl.BlockSpec((1, d, dh), lambda h: (h, 0, 0)),
            pl.BlockSpec((1, d, dh), lambda h: (h, 0, 0)),
            pl.BlockSpec((1, dh, d), lambda h: (h, 0, 0)),
            pl.BlockSpec((dh, IDX_DIM_), lambda h: (0, 0)),
            pl.BlockSpec((dh, IDX_DIM_), lambda h: (0, 0)),
        ],
        out_specs=pl.BlockSpec((t, d), lambda h: (0, 0)),
        out_shape=jax.ShapeDtypeStruct((t, d), jnp.float32),
        scratch_shapes=[pltpu.VMEM((t, t), jnp.int32)],
        compiler_params=pltpu.CompilerParams(
            dimension_semantics=("arbitrary",)),
    )(x, wq_h, wk_h, wv_h, wo_h, Wq_idx, Wk_idx)
    return out.reshape(b, t, d)
